# Initial kernel scaffold; baseline (speedup 1.0000x reference)
import jax
import jax.numpy as jnp
from jax.experimental import pallas as pl


def kernel(x, edge_index, edge_attr, batch, W_node, b_node, W1, b1, W2, b2, W3, b3, W_post, b_post):
    raise NotImplementedError("write your pallas kernel here")


# speedup vs baseline: 15.7962x; 15.7962x over previous
import functools

import jax
import jax.numpy as jnp
from jax import lax
from jax.experimental import pallas as pl
from jax.experimental.pallas import tpu as pltpu
from jax.experimental.pallas import tpu_sc as plsc

N = 10000
E = 320000
D_IN = 128
H = 64
C = 32

NW = 32
K = 128
NCH = 80
E_PAD = NW * NCH * K
NP = 10240
RPT = NP // 16
PAD_ROWS = NP - N
R = 2048


def _mesh():
    return plsc.VectorSubcoreMesh(core_axis_name="c", subcore_axis_name="s")


@functools.partial(
    pl.kernel,
    mesh=_mesh(),
    out_type=jax.ShapeDtypeStruct((NW, NP // 16, 16), jnp.float32),
    scratch_types=[
        pltpu.VMEM((NCH, K), jnp.int32),
        pltpu.VMEM((NP // 16, 16), jnp.float32),
    ],
    compiler_params=pltpu.CompilerParams(needs_layout_passes=False),
)
def _deg_kernel(dst_hbm, degp_hbm, dstv, degv):
    c = lax.axis_index("c")
    s = lax.axis_index("s")
    wid = s * 2 + c
    zeros = jnp.zeros((16,), jnp.float32)

    def zbody(i, carry):
        degv[i, :] = zeros
        return carry

    lax.fori_loop(0, NP // 16, zbody, 0)
    pltpu.sync_copy(dst_hbm.at[wid], dstv)
    ones = jnp.ones((16,), jnp.float32)

    def cbody(j, carry):
        for k in range(K // 16):
            idx = dstv[j, pl.ds(k * 16, 16)]
            plsc.addupdate_scatter(degv, [idx >> 4, idx & 15], ones)
        return carry

    lax.fori_loop(0, NCH, cbody, 0)
    pltpu.sync_copy(degv, degp_hbm.at[wid])


@functools.partial(
    pl.kernel,
    mesh=_mesh(),
    out_type=jax.ShapeDtypeStruct((2, NP, H), jnp.float32),
    scratch_types=[
        pltpu.VMEM((NCH, K), jnp.int32),
        pltpu.VMEM((NCH, K), jnp.int32),
        pltpu.VMEM((2, K, H), jnp.float32),
        pltpu.VMEM_SHARED((NP, H), jnp.float32),
        pltpu.SemaphoreType.DMA,
        pltpu.SemaphoreType.DMA,
    ],
    compiler_params=pltpu.CompilerParams(
        needs_layout_passes=False, use_tc_tiling_on_sc=False
    ),
)
def _agg_kernel(g_hbm, src_hbm, dst_hbm, part_hbm, srcv, dstv, rows, acc, sem0, sem1):
    c = lax.axis_index("c")
    s = lax.axis_index("s")
    wid = s * 2 + c
    pltpu.async_copy(src_hbm.at[wid], srcv, sem0)
    pltpu.async_copy(dst_hbm.at[wid], dstv, sem1)
    pltpu.sync_copy(g_hbm.at[pl.ds(s * RPT, RPT)], acc.at[pl.ds(s * RPT, RPT)])
    pltpu.make_async_copy(src_hbm.at[wid], srcv, sem0).wait()
    pltpu.make_async_copy(dst_hbm.at[wid], dstv, sem1).wait()
    plsc.subcore_barrier()

    sems = (sem0, sem1)
    pltpu.async_copy(g_hbm.at[srcv.at[0]], rows.at[0], sem0)

    def body(i, carry):
        for b in range(2):
            j = i * 2 + b
            nb = 1 - b

            @pl.when(j + 1 < NCH)
            def _prefetch():
                pltpu.async_copy(g_hbm.at[srcv.at[j + 1]], rows.at[nb], sems[nb])

            pltpu.make_async_copy(g_hbm.at[srcv.at[j]], rows.at[b], sems[b]).wait()
            pltpu.sync_copy(rows.at[b], acc.at[dstv.at[j]], add=True)
        return carry

    lax.fori_loop(0, NCH // 2, body, 0)
    plsc.subcore_barrier()
    pltpu.sync_copy(acc.at[pl.ds(s * RPT, RPT)], part_hbm.at[c].at[pl.ds(s * RPT, RPT)])


def _tc_prologue(x_p, degp, wtn, bn, wt1):
    def body(x_b, degp_b, wtn_b, bn_b, wt1_b, g1_b, dinv_b):
        deg = jnp.sum(degp_b[...], axis=0)[:, None] + 1.0
        dinv = lax.rsqrt(deg)
        h0 = jnp.dot(x_b[...], wtn_b[...], preferred_element_type=jnp.float32) + bn_b[...]
        g1_b[...] = dinv * jnp.dot(h0, wt1_b[...], preferred_element_type=jnp.float32)
        dinv_b[...] = jnp.broadcast_to(dinv, (R, H))

    return pl.pallas_call(
        body,
        grid=(NP // R,),
        in_specs=[
            pl.BlockSpec((R, D_IN), lambda i: (i, 0)),
            pl.BlockSpec((NW, R), lambda i: (0, i)),
            pl.BlockSpec((D_IN, H), lambda i: (0, 0)),
            pl.BlockSpec((1, H), lambda i: (0, 0)),
            pl.BlockSpec((H, H), lambda i: (0, 0)),
        ],
        out_specs=[
            pl.BlockSpec((R, H), lambda i: (i, 0)),
            pl.BlockSpec((R, H), lambda i: (i, 0)),
        ],
        out_shape=[
            jax.ShapeDtypeStruct((NP, H), jnp.float32),
            jax.ShapeDtypeStruct((NP, H), jnp.float32),
        ],
    )(x_p, degp, wtn, bn, wt1)


def _tc_combine(parts, g, dinv64, b, wt):
    def body(p_b, g_b, d_b, b_b, wt_b, o_b):
        sagg = p_b[0] + p_b[1] - g_b[...]
        h = jnp.maximum(d_b[...] * sagg + b_b[...], 0.0)
        o_b[...] = d_b[...] * jnp.dot(h, wt_b[...], preferred_element_type=jnp.float32)

    return pl.pallas_call(
        body,
        grid=(NP // R,),
        in_specs=[
            pl.BlockSpec((2, R, H), lambda i: (0, i, 0)),
            pl.BlockSpec((R, H), lambda i: (i, 0)),
            pl.BlockSpec((R, H), lambda i: (i, 0)),
            pl.BlockSpec((1, H), lambda i: (0, 0)),
            pl.BlockSpec((H, H), lambda i: (0, 0)),
        ],
        out_specs=pl.BlockSpec((R, H), lambda i: (i, 0)),
        out_shape=jax.ShapeDtypeStruct((NP, H), jnp.float32),
    )(parts, g, dinv64, b, wt)


def _tc_epilogue(parts, g, dinv64, b3, wtp, bp):
    def body(p_b, g_b, d_b, b_b, wtp_b, bp_b, o_b):
        sagg = p_b[0] + p_b[1] - g_b[...]
        h = jnp.maximum(d_b[...] * sagg + b_b[...], 0.0)
        o_b[...] = jnp.dot(h, wtp_b[...], preferred_element_type=jnp.float32) + bp_b[...]

    return pl.pallas_call(
        body,
        grid=(NP // R,),
        in_specs=[
            pl.BlockSpec((2, R, H), lambda i: (0, i, 0)),
            pl.BlockSpec((R, H), lambda i: (i, 0)),
            pl.BlockSpec((R, H), lambda i: (i, 0)),
            pl.BlockSpec((1, H), lambda i: (0, 0)),
            pl.BlockSpec((H, C), lambda i: (0, 0)),
            pl.BlockSpec((1, C), lambda i: (0, 0)),
        ],
        out_specs=pl.BlockSpec((R, C), lambda i: (i, 0)),
        out_shape=jax.ShapeDtypeStruct((NP, C), jnp.float32),
    )(parts, g, dinv64, b3, wtp, bp)


def kernel(x, edge_index, edge_attr, batch, W_node, b_node, W1, b1, W2, b2, W3, b3, W_post, b_post):
    del edge_attr, batch
    src = edge_index[0]
    dst = edge_index[1]
    pad_e = E_PAD - E
    pad_src = jnp.zeros((pad_e,), jnp.int32)
    pad_dst = N + (jnp.arange(pad_e, dtype=jnp.int32) % PAD_ROWS)
    src_p = jnp.concatenate([src, pad_src]).reshape(NW, NCH, K)
    dst_p = jnp.concatenate([dst, pad_dst]).reshape(NW, NCH, K)
    x_p = jnp.pad(x, ((0, NP - N), (0, 0)))

    degp = _deg_kernel(dst_p).reshape(NW, NP)
    g1, dinv64 = _tc_prologue(x_p, degp, W_node.T, b_node[None], W1.T)
    parts1 = _agg_kernel(g1, src_p, dst_p)
    g2 = _tc_combine(parts1, g1, dinv64, b1[None], W2.T)
    parts2 = _agg_kernel(g2, src_p, dst_p)
    g3 = _tc_combine(parts2, g2, dinv64, b2[None], W3.T)
    parts3 = _agg_kernel(g3, src_p, dst_p)
    out = _tc_epilogue(parts3, g3, dinv64, b3[None], W_post.T, b_post[None])
    return out[:N]


# speedup vs baseline: 15.8653x; 1.0044x over previous
import functools

import jax
import jax.numpy as jnp
from jax import lax
from jax.experimental import pallas as pl
from jax.experimental.pallas import tpu as pltpu
from jax.experimental.pallas import tpu_sc as plsc

N = 10000
E = 320000
D_IN = 128
H = 64
C = 32

NW = 32
K = 128
NCH = 80
E_PAD = NW * NCH * K
NP = 10240
RPT = NP // 16
PAD_ROWS = NP - N
R = 2048
NBUF = 4
DEPTH = 2


def _mesh():
    return plsc.VectorSubcoreMesh(core_axis_name="c", subcore_axis_name="s")


@functools.partial(
    pl.kernel,
    mesh=_mesh(),
    out_type=jax.ShapeDtypeStruct((NW, NP // 16, 16), jnp.float32),
    scratch_types=[
        pltpu.VMEM((NCH, K), jnp.int32),
        pltpu.VMEM((NP // 16, 16), jnp.float32),
    ],
    compiler_params=pltpu.CompilerParams(needs_layout_passes=False),
)
def _deg_kernel(dst_hbm, degp_hbm, dstv, degv):
    c = lax.axis_index("c")
    s = lax.axis_index("s")
    wid = s * 2 + c
    zeros = jnp.zeros((16,), jnp.float32)

    def zbody(i, carry):
        degv[i, :] = zeros
        return carry

    lax.fori_loop(0, NP // 16, zbody, 0)
    pltpu.sync_copy(dst_hbm.at[wid], dstv)
    ones = jnp.ones((16,), jnp.float32)

    def cbody(j, carry):
        for k in range(K // 16):
            idx = dstv[j, pl.ds(k * 16, 16)]
            plsc.addupdate_scatter(degv, [idx >> 4, idx & 15], ones)
        return carry

    lax.fori_loop(0, NCH, cbody, 0)
    pltpu.sync_copy(degv, degp_hbm.at[wid])


@functools.partial(
    pl.kernel,
    mesh=_mesh(),
    out_type=jax.ShapeDtypeStruct((2, NP, H), jnp.float32),
    scratch_types=[
        pltpu.VMEM((NCH, K), jnp.int32),
        pltpu.VMEM((NCH, K), jnp.int32),
        pltpu.VMEM((NBUF, K, H), jnp.float32),
        pltpu.VMEM_SHARED((NP, H), jnp.float32),
        [pltpu.SemaphoreType.DMA] * NBUF,
        [pltpu.SemaphoreType.DMA] * NBUF,
    ],
    compiler_params=pltpu.CompilerParams(
        needs_layout_passes=False, use_tc_tiling_on_sc=False
    ),
)
def _agg_kernel(g_hbm, src_hbm, dst_hbm, part_hbm, srcv, dstv, rows, acc, gsem, ssem):
    c = lax.axis_index("c")
    s = lax.axis_index("s")
    wid = s * 2 + c
    pltpu.async_copy(src_hbm.at[wid], srcv, gsem[0])
    pltpu.async_copy(dst_hbm.at[wid], dstv, gsem[1])
    pltpu.sync_copy(g_hbm.at[pl.ds(s * RPT, RPT)], acc.at[pl.ds(s * RPT, RPT)])
    pltpu.make_async_copy(src_hbm.at[wid], srcv, gsem[0]).wait()
    pltpu.make_async_copy(dst_hbm.at[wid], dstv, gsem[1]).wait()
    plsc.subcore_barrier()

    for j in range(DEPTH):
        pltpu.async_copy(g_hbm.at[srcv.at[j]], rows.at[j], gsem[j])

    def body(i4, carry):
        for u in range(NBUF):
            j = i4 * NBUF + u
            b = u
            bg = (u + DEPTH) % NBUF
            jg = j + DEPTH

            @pl.when(j >= NBUF - DEPTH)
            def _free():
                pltpu.make_async_copy(
                    rows.at[bg], acc.at[dstv.at[j]], ssem[bg]
                ).wait()

            @pl.when(jg < NCH)
            def _prefetch():
                pltpu.async_copy(g_hbm.at[srcv.at[jg]], rows.at[bg], gsem[bg])

            pltpu.make_async_copy(g_hbm.at[srcv.at[j]], rows.at[b], gsem[b]).wait()
            pltpu.async_copy(rows.at[b], acc.at[dstv.at[j]], ssem[b], add=True)
        return carry

    lax.fori_loop(0, NCH // NBUF, body, 0)
    for j in range(NCH - DEPTH, NCH):
        b = j % NBUF
        pltpu.make_async_copy(rows.at[b], acc.at[dstv.at[j]], ssem[b]).wait()
    plsc.subcore_barrier()
    pltpu.sync_copy(acc.at[pl.ds(s * RPT, RPT)], part_hbm.at[c].at[pl.ds(s * RPT, RPT)])


def _tc_prologue(x_p, degp, wtn, bn, wt1):
    def body(x_b, degp_b, wtn_b, bn_b, wt1_b, g1_b, dinv_b):
        deg = jnp.sum(degp_b[...], axis=0)[:, None] + 1.0
        dinv = lax.rsqrt(deg)
        h0 = jnp.dot(x_b[...], wtn_b[...], preferred_element_type=jnp.float32) + bn_b[...]
        g1_b[...] = dinv * jnp.dot(h0, wt1_b[...], preferred_element_type=jnp.float32)
        dinv_b[...] = jnp.broadcast_to(dinv, (R, H))

    return pl.pallas_call(
        body,
        grid=(NP // R,),
        in_specs=[
            pl.BlockSpec((R, D_IN), lambda i: (i, 0)),
            pl.BlockSpec((NW, R), lambda i: (0, i)),
            pl.BlockSpec((D_IN, H), lambda i: (0, 0)),
            pl.BlockSpec((1, H), lambda i: (0, 0)),
            pl.BlockSpec((H, H), lambda i: (0, 0)),
        ],
        out_specs=[
            pl.BlockSpec((R, H), lambda i: (i, 0)),
            pl.BlockSpec((R, H), lambda i: (i, 0)),
        ],
        out_shape=[
            jax.ShapeDtypeStruct((NP, H), jnp.float32),
            jax.ShapeDtypeStruct((NP, H), jnp.float32),
        ],
    )(x_p, degp, wtn, bn, wt1)


def _tc_combine(parts, g, dinv64, b, wt):
    def body(p_b, g_b, d_b, b_b, wt_b, o_b):
        sagg = p_b[0] + p_b[1] - g_b[...]
        h = jnp.maximum(d_b[...] * sagg + b_b[...], 0.0)
        o_b[...] = d_b[...] * jnp.dot(h, wt_b[...], preferred_element_type=jnp.float32)

    return pl.pallas_call(
        body,
        grid=(NP // R,),
        in_specs=[
            pl.BlockSpec((2, R, H), lambda i: (0, i, 0)),
            pl.BlockSpec((R, H), lambda i: (i, 0)),
            pl.BlockSpec((R, H), lambda i: (i, 0)),
            pl.BlockSpec((1, H), lambda i: (0, 0)),
            pl.BlockSpec((H, H), lambda i: (0, 0)),
        ],
        out_specs=pl.BlockSpec((R, H), lambda i: (i, 0)),
        out_shape=jax.ShapeDtypeStruct((NP, H), jnp.float32),
    )(parts, g, dinv64, b, wt)


def _tc_epilogue(parts, g, dinv64, b3, wtp, bp):
    def body(p_b, g_b, d_b, b_b, wtp_b, bp_b, o_b):
        sagg = p_b[0] + p_b[1] - g_b[...]
        h = jnp.maximum(d_b[...] * sagg + b_b[...], 0.0)
        o_b[...] = jnp.dot(h, wtp_b[...], preferred_element_type=jnp.float32) + bp_b[...]

    return pl.pallas_call(
        body,
        grid=(NP // R,),
        in_specs=[
            pl.BlockSpec((2, R, H), lambda i: (0, i, 0)),
            pl.BlockSpec((R, H), lambda i: (i, 0)),
            pl.BlockSpec((R, H), lambda i: (i, 0)),
            pl.BlockSpec((1, H), lambda i: (0, 0)),
            pl.BlockSpec((H, C), lambda i: (0, 0)),
            pl.BlockSpec((1, C), lambda i: (0, 0)),
        ],
        out_specs=pl.BlockSpec((R, C), lambda i: (i, 0)),
        out_shape=jax.ShapeDtypeStruct((NP, C), jnp.float32),
    )(parts, g, dinv64, b3, wtp, bp)


def kernel(x, edge_index, edge_attr, batch, W_node, b_node, W1, b1, W2, b2, W3, b3, W_post, b_post):
    del edge_attr, batch
    src = edge_index[0]
    dst = edge_index[1]
    pad_e = E_PAD - E
    pad_src = jnp.zeros((pad_e,), jnp.int32)
    pad_dst = N + (jnp.arange(pad_e, dtype=jnp.int32) % PAD_ROWS)
    src_p = jnp.concatenate([src, pad_src]).reshape(NW, NCH, K)
    dst_p = jnp.concatenate([dst, pad_dst]).reshape(NW, NCH, K)
    x_p = jnp.pad(x, ((0, NP - N), (0, 0)))

    degp = _deg_kernel(dst_p).reshape(NW, NP)
    g1, dinv64 = _tc_prologue(x_p, degp, W_node.T, b_node[None], W1.T)
    parts1 = _agg_kernel(g1, src_p, dst_p)
    g2 = _tc_combine(parts1, g1, dinv64, b1[None], W2.T)
    parts2 = _agg_kernel(g2, src_p, dst_p)
    g3 = _tc_combine(parts2, g2, dinv64, b2[None], W3.T)
    parts3 = _agg_kernel(g3, src_p, dst_p)
    out = _tc_epilogue(parts3, g3, dinv64, b3[None], W_post.T, b_post[None])
    return out[:N]
